# baseline (speedup 1.0000x reference)
import functools

import jax
import jax.numpy as jnp
from jax import lax
from jax.experimental import pallas as pl
from jax.experimental.pallas import tpu as pltpu
from jax.experimental.pallas import tpu_sc as plsc

BATCH = 16384
EMBED_DIM = 32
LANES = 16
NUM_CORES = 2
NUM_SUBCORES = 16
NUM_WORKERS = NUM_CORES * NUM_SUBCORES
B_PER_W = BATCH // NUM_WORKERS
CHUNK = 128
NCHUNKS = B_PER_W // CHUNK
GROUPS = B_PER_W // LANES


def _gmf_body(users_hbm, items_hbm, utab_hbm, itab_hbm, params_hbm, out_hbm,
              uidx_v, iidx_v, urows_v, irows_v, wv, outv, sem_u, sem_i,
              sem_s):
    wid = lax.axis_index("s") * NUM_CORES + lax.axis_index("c")
    base = wid * B_PER_W

    for j in range(NCHUNKS):
        pltpu.sync_copy(users_hbm.at[pl.ds(base + j * CHUNK, CHUNK)],
                        uidx_v.at[j])
        pltpu.sync_copy(items_hbm.at[pl.ds(base + j * CHUNK, CHUNK)],
                        iidx_v.at[j])
    pltpu.sync_copy(params_hbm, wv)

    copies = []
    for j in range(NCHUNKS):
        copies.append(pltpu.async_copy(
            utab_hbm.at[uidx_v.at[j]],
            urows_v.at[pl.ds(j * CHUNK, CHUNK)], sem_u))
        copies.append(pltpu.async_copy(
            itab_hbm.at[iidx_v.at[j]],
            irows_v.at[pl.ds(j * CHUNK, CHUNK)], sem_i))
    for c in copies:
        c.wait()

    w_lo = wv[pl.ds(0, LANES)]
    w_hi = wv[pl.ds(LANES, LANES)]
    bias = wv[pl.ds(2 * LANES, LANES)][0]
    lane = lax.iota(jnp.int32, LANES)

    def group(g, carry):
        rowids = g * LANES + lane
        acc = jnp.full((LANES,), bias, dtype=jnp.float32)
        for d in range(EMBED_DIM):
            col = jnp.full((LANES,), d, dtype=jnp.int32)
            uc = plsc.load_gather(urows_v, [rowids, col])
            ic = plsc.load_gather(irows_v, [rowids, col])
            wd = w_lo[d] if d < LANES else w_hi[d - LANES]
            acc = acc + uc * ic * wd
        outv[pl.ds(g * LANES, LANES)] = acc
        return carry

    lax.fori_loop(0, GROUPS, group, 0)

    pltpu.sync_copy(outv, out_hbm.at[pl.ds(base, B_PER_W)])


@jax.jit
def _gmf(users, items, user_table, item_table, params):
    mesh = plsc.VectorSubcoreMesh(core_axis_name="c", subcore_axis_name="s")
    return pl.kernel(
        _gmf_body,
        out_type=jax.ShapeDtypeStruct((BATCH,), jnp.float32),
        mesh=mesh,
        compiler_params=pltpu.CompilerParams(
            needs_layout_passes=False, use_tc_tiling_on_sc=False),
        scratch_types=[
            pltpu.VMEM((NCHUNKS, CHUNK), jnp.int32),
            pltpu.VMEM((NCHUNKS, CHUNK), jnp.int32),
            pltpu.VMEM((B_PER_W, EMBED_DIM), jnp.float32),
            pltpu.VMEM((B_PER_W, EMBED_DIM), jnp.float32),
            pltpu.VMEM((48,), jnp.float32),
            pltpu.VMEM((B_PER_W,), jnp.float32),
            pltpu.SemaphoreType.DMA,
            pltpu.SemaphoreType.DMA,
            pltpu.SemaphoreType.DMA,
        ],
    )(users, items, user_table, item_table, params)


def kernel(users, items, user_table, item_table, W, b):
    params = jnp.zeros((48,), jnp.float32)
    params = params.at[:EMBED_DIM].set(W.reshape(-1))
    params = params.at[EMBED_DIM:EMBED_DIM + 1].set(b)
    return _gmf(users, items, user_table, item_table, params)
